# initial kernel scaffold (unmeasured)
import jax
import jax.numpy as jnp
from jax import lax
from jax.experimental import pallas as pl
from jax.experimental.pallas import tpu as pltpu

NEG_INF = -1e30


def kernel(Q, K, V, bt, lens):
    B, _, H, D = Q.shape
    P, BS, _, _ = K.shape
    NB = bt.shape[1]
    scale = D ** -0.5

    Q2 = Q.reshape(B, H, D)
    lens2 = lens.reshape(B, 1)

    def body(q_ref, k_ref, v_ref, bt_ref, lens_ref, out_ref,
             send_ref, recv_ref, send_sem, recv_sem):
        my_x = lax.axis_index("x")
        peer = (1 - my_x, lax.axis_index("y"), lax.axis_index("z"))

        barrier_sem = pltpu.get_barrier_semaphore()
        pl.semaphore_signal(barrier_sem, inc=1, device_id=peer,
                            device_id_type=pl.DeviceIdType.MESH)
        pl.semaphore_wait(barrier_sem, 1)

        q = q_ref[...]
        k = k_ref[...]
        v = v_ref[...]
        btl = bt_ref[...] - my_x * P
        j_iota = lax.broadcasted_iota(jnp.int32, (B, NB), 1)
        valid = (j_iota < lens_ref[...]) & (btl >= 0) & (btl < P)
        p_iota = lax.broadcasted_iota(jnp.int32, (B, NB, P), 2)
        M = jnp.where((btl[:, :, None] == p_iota) & valid[:, :, None],
                      1.0, 0.0).astype(jnp.float32)

        s_all = jnp.einsum("bhd,pthd->bhpt", q, k,
                           preferred_element_type=jnp.float32) * scale
        s_g = jnp.einsum("bjp,bhpt->bhjt", M, s_all,
                         preferred_element_type=jnp.float32)
        s_g = jnp.where(valid[:, None, :, None], s_g, NEG_INF)
        m = jnp.max(s_g, axis=(2, 3))
        p_exp = jnp.exp(s_g - m[:, :, None, None])
        s_sum = jnp.sum(p_exp, axis=(2, 3))
        w = jnp.einsum("bjp,bhjt->bhpt", M, p_exp,
                       preferred_element_type=jnp.float32)
        o = jnp.einsum("bhpt,pthd->bhd", w, v,
                       preferred_element_type=jnp.float32)

        send_ref[...] = jnp.concatenate(
            [o, m[:, :, None], s_sum[:, :, None]], axis=-1)

        rdma = pltpu.make_async_remote_copy(
            src_ref=send_ref, dst_ref=recv_ref,
            send_sem=send_sem, recv_sem=recv_sem,
            device_id=peer, device_id_type=pl.DeviceIdType.MESH)
        rdma.start()
        rdma.wait()

        r = recv_ref[...]
        o_p, m_p, s_p = r[:, :, :D], r[:, :, D], r[:, :, D + 1]
        m_tot = jnp.maximum(m, m_p)
        a = jnp.exp(m - m_tot)
        a_p = jnp.exp(m_p - m_tot)
        num = o * a[:, :, None] + o_p * a_p[:, :, None]
        den = s_sum * a + s_p * a_p
        out_ref[...] = num / den[:, :, None]

    out = pl.pallas_call(
        body,
        out_shape=jax.ShapeDtypeStruct((B, H, D), jnp.float32),
        in_specs=[pl.BlockSpec(memory_space=pltpu.VMEM)] * 5,
        out_specs=pl.BlockSpec(memory_space=pltpu.VMEM),
        scratch_shapes=[
            pltpu.VMEM((B, H, D + 2), jnp.float32),
            pltpu.VMEM((B, H, D + 2), jnp.float32),
            pltpu.SemaphoreType.DMA,
            pltpu.SemaphoreType.DMA,
        ],
        compiler_params=pltpu.CompilerParams(collective_id=0),
    )(Q2, K, V, bt, lens2)
    return out.reshape(B, 1, H, D)


# baseline (device time: 54439 ns/iter reference)
import jax
import jax.numpy as jnp
from jax import lax
from jax.experimental import pallas as pl
from jax.experimental.pallas import tpu as pltpu

NEG_INF = -1e30


def kernel(Q, K, V, bt, lens):
    B, _, H, D = Q.shape
    P, BS, _, _ = K.shape
    NB = bt.shape[1]
    scale = D ** -0.5

    Q2 = Q.reshape(B, H, D)
    lens2 = lens.reshape(B, 1)

    T = P * BS

    def body(q_ref, k_ref, v_ref, bt_ref, lens_ref, out_ref,
             send_ref, recv_ref, send_sem, recv_sem):
        my_x = lax.axis_index("x")
        peer = (1 - my_x, lax.axis_index("y"), lax.axis_index("z"))

        barrier_sem = pltpu.get_barrier_semaphore()
        pl.semaphore_signal(barrier_sem, inc=1, device_id=peer,
                            device_id_type=pl.DeviceIdType.MESH)
        pl.semaphore_wait(barrier_sem, 1)

        q = q_ref[...]
        ktok = k_ref[...].reshape(T, H, D)
        vtok = v_ref[...].reshape(T, H, D)
        btl = bt_ref[...] - my_x * P
        j_iota = lax.broadcasted_iota(jnp.int32, (B, NB), 1)
        valid = (j_iota < lens_ref[...]) & (btl >= 0) & (btl < P)

        r_i = lax.broadcasted_iota(jnp.int32, (T, NB), 0)
        j_i = lax.broadcasted_iota(jnp.int32, (T, NB), 1)
        R = (r_i // BS == j_i).astype(jnp.float32)
        pc_f = (lax.broadcasted_iota(jnp.int32, (T, T), 1) // BS
                ).astype(jnp.float32)
        tc_f = (lax.broadcasted_iota(jnp.int32, (T, T), 1) % BS
                ).astype(jnp.float32)
        t_col = (lax.broadcasted_iota(jnp.int32, (T, 1), 0) % BS
                 ).astype(jnp.float32)
        btl_cols = jax.lax.dot_general(
            R, btl.astype(jnp.float32).T, (((1,), (0,)), ((), ())),
            preferred_element_type=jnp.float32)
        val_cols = jax.lax.dot_general(
            R, valid.astype(jnp.float32).T, (((1,), (0,)), ((), ())),
            preferred_element_type=jnp.float32)

        o_parts, m_parts, s_parts = [], [], []
        for b in range(B):
            btl_c = btl_cols[:, b:b + 1]
            val_c = val_cols[:, b:b + 1]
            G = ((btl_c == pc_f) & (t_col == tc_f) & (val_c > 0.5)
                 ).astype(jnp.float32)
            s_ht = jnp.sum(ktok * q[b][None, :, :], axis=2) * scale
            s_g = jax.lax.dot_general(
                G, s_ht, (((1,), (0,)), ((), ())),
                preferred_element_type=jnp.float32)
            s_g = jnp.where(val_c > 0.5, s_g, NEG_INF)
            m_b = jnp.max(s_g, axis=0, keepdims=True)
            p_exp = jnp.exp(s_g - m_b)
            s_b = jnp.sum(p_exp, axis=0, keepdims=True)
            w_tok = jax.lax.dot_general(
                G, p_exp, (((0,), (0,)), ((), ())),
                preferred_element_type=jnp.float32)
            o_b = jnp.sum(w_tok[:, :, None] * vtok, axis=0)
            o_parts.append(o_b[None])
            m_parts.append(m_b)
            s_parts.append(s_b)

        o = jnp.concatenate(o_parts, axis=0)
        m = jnp.concatenate(m_parts, axis=0)
        s_sum = jnp.concatenate(s_parts, axis=0)

        send_ref[...] = jnp.concatenate(
            [o, m[:, :, None], s_sum[:, :, None]], axis=-1)

        rdma = pltpu.make_async_remote_copy(
            src_ref=send_ref, dst_ref=recv_ref,
            send_sem=send_sem, recv_sem=recv_sem,
            device_id=peer, device_id_type=pl.DeviceIdType.MESH)
        rdma.start()
        rdma.wait()

        r = recv_ref[...]
        o_p, m_p, s_p = r[:, :, :D], r[:, :, D], r[:, :, D + 1]
        m_tot = jnp.maximum(m, m_p)
        a = jnp.exp(m - m_tot)
        a_p = jnp.exp(m_p - m_tot)
        num = o * a[:, :, None] + o_p * a_p[:, :, None]
        den = s_sum * a + s_p * a_p
        out_ref[...] = num / den[:, :, None]

    out = pl.pallas_call(
        body,
        out_shape=jax.ShapeDtypeStruct((B, H, D), jnp.float32),
        in_specs=[pl.BlockSpec(memory_space=pltpu.VMEM)] * 5,
        out_specs=pl.BlockSpec(memory_space=pltpu.VMEM),
        scratch_shapes=[
            pltpu.VMEM((B, H, D + 2), jnp.float32),
            pltpu.VMEM((B, H, D + 2), jnp.float32),
            pltpu.SemaphoreType.DMA,
            pltpu.SemaphoreType.DMA,
        ],
        compiler_params=pltpu.CompilerParams(
            collective_id=0, vmem_limit_bytes=96 * 1024 * 1024),
    )(Q2, K, V, bt, lens2)
    return out.reshape(B, 1, H, D)


# device time: 31708 ns/iter; 1.7169x vs baseline; 1.7169x over previous
import jax
import jax.numpy as jnp
from jax import lax
from jax.experimental import pallas as pl
from jax.experimental.pallas import tpu as pltpu

NEG_INF = -1e30


def kernel(Q, K, V, bt, lens):
    B, _, H, D = Q.shape
    P, BS, _, _ = K.shape
    NB = bt.shape[1]
    scale = D ** -0.5

    Q2 = Q.reshape(B, H, D)
    lens2 = lens.reshape(B, 1)

    T = P * BS

    def body(q_ref, k_ref, v_ref, bt_ref, lens_ref, out_ref,
             send_ref, recv_ref, send_sem, recv_sem):
        my_x = lax.axis_index("x")
        peer = (1 - my_x, lax.axis_index("y"), lax.axis_index("z"))

        barrier_sem = pltpu.get_barrier_semaphore()
        pl.semaphore_signal(barrier_sem, inc=1, device_id=peer,
                            device_id_type=pl.DeviceIdType.MESH)
        pl.semaphore_wait(barrier_sem, 1)

        q = q_ref[...]
        ktok = k_ref[...].reshape(T, H, D)
        vtok = v_ref[...].reshape(T, H, D)
        btl = bt_ref[...] - my_x * P
        j_iota = lax.broadcasted_iota(jnp.int32, (B, NB), 1)
        valid = (j_iota < lens_ref[...]) & (btl >= 0) & (btl < P)

        btl_v = jnp.where(valid, btl, -1)
        p_idx = lax.broadcasted_iota(jnp.int32, (B, P, NB), 1)
        hits = (btl_v[:, None, :] == p_idx).astype(jnp.float32)
        cnt = jnp.sum(hits, axis=2)
        r_i = lax.broadcasted_iota(jnp.int32, (T, P), 0)
        p_col = lax.broadcasted_iota(jnp.int32, (T, P), 1)
        R = (r_i // BS == p_col).astype(jnp.float32)
        cnt_tok = jax.lax.dot_general(
            R, cnt.T, (((1,), (0,)), ((), ())),
            preferred_element_type=jnp.float32)

        o_parts, m_parts, s_parts = [], [], []
        for b in range(B):
            ct = cnt_tok[:, b:b + 1]
            s_ht = jnp.sum(ktok * q[b][None, :, :], axis=2) * scale
            s_msk = jnp.where(ct > 0.5, s_ht, NEG_INF)
            m_b = jnp.max(s_msk, axis=0, keepdims=True)
            p_exp = jnp.exp(s_msk - m_b) * ct
            s_b = jnp.sum(p_exp, axis=0, keepdims=True)
            o_b = jnp.sum(p_exp[:, :, None] * vtok, axis=0)
            o_parts.append(o_b[None])
            m_parts.append(m_b)
            s_parts.append(s_b)

        o = jnp.concatenate(o_parts, axis=0)
        m = jnp.concatenate(m_parts, axis=0)
        s_sum = jnp.concatenate(s_parts, axis=0)

        send_ref[...] = jnp.concatenate(
            [o, m[:, :, None], s_sum[:, :, None]], axis=-1)

        rdma = pltpu.make_async_remote_copy(
            src_ref=send_ref, dst_ref=recv_ref,
            send_sem=send_sem, recv_sem=recv_sem,
            device_id=peer, device_id_type=pl.DeviceIdType.MESH)
        rdma.start()
        rdma.wait()

        r = recv_ref[...]
        o_p, m_p, s_p = r[:, :, :D], r[:, :, D], r[:, :, D + 1]
        m_tot = jnp.maximum(m, m_p)
        a = jnp.exp(m - m_tot)
        a_p = jnp.exp(m_p - m_tot)
        num = o * a[:, :, None] + o_p * a_p[:, :, None]
        den = s_sum * a + s_p * a_p
        out_ref[...] = num / den[:, :, None]

    out = pl.pallas_call(
        body,
        out_shape=jax.ShapeDtypeStruct((B, H, D), jnp.float32),
        in_specs=[pl.BlockSpec(memory_space=pltpu.VMEM)] * 5,
        out_specs=pl.BlockSpec(memory_space=pltpu.VMEM),
        scratch_shapes=[
            pltpu.VMEM((B, H, D + 2), jnp.float32),
            pltpu.VMEM((B, H, D + 2), jnp.float32),
            pltpu.SemaphoreType.DMA,
            pltpu.SemaphoreType.DMA,
        ],
        compiler_params=pltpu.CompilerParams(
            collective_id=0, vmem_limit_bytes=96 * 1024 * 1024),
    )(Q2, K, V, bt, lens2)
    return out.reshape(B, 1, H, D)


# device time: 16278 ns/iter; 3.3443x vs baseline; 1.9479x over previous
import jax
import jax.numpy as jnp
from jax import lax
from jax.experimental import pallas as pl
from jax.experimental.pallas import tpu as pltpu

NEG_INF = -1e30


def kernel(Q, K, V, bt, lens):
    B, _, H, D = Q.shape
    P, BS, _, _ = K.shape
    NB = bt.shape[1]
    scale = D ** -0.5

    Q2 = Q.reshape(B, H, D)
    lens2 = lens.reshape(B, 1)

    T = P * BS

    def body(q_ref, k_ref, v_ref, bt_ref, lens_ref, out_ref,
             send_ref, recv_ref, send_sem, recv_sem):
        my_x = lax.axis_index("x")
        peer = (1 - my_x, lax.axis_index("y"), lax.axis_index("z"))

        barrier_sem = pltpu.get_barrier_semaphore()
        pl.semaphore_signal(barrier_sem, inc=1, device_id=peer,
                            device_id_type=pl.DeviceIdType.MESH)
        pl.semaphore_wait(barrier_sem, 1)

        q = q_ref[...]
        btl = bt_ref[...] - my_x * P
        j_iota = lax.broadcasted_iota(jnp.int32, (B, NB), 1)
        valid = (j_iota < lens_ref[...]) & (btl >= 0) & (btl < P)

        btl_v = jnp.where(valid, btl, -1)
        p_idx = lax.broadcasted_iota(jnp.int32, (B, P, NB), 1)
        hits = (btl_v[:, None, :] == p_idx).astype(jnp.float32)
        cnt = jnp.sum(hits, axis=2)
        r_i = lax.broadcasted_iota(jnp.int32, (T, P), 0)
        p_col = lax.broadcasted_iota(jnp.int32, (T, P), 1)
        R = (r_i // BS == p_col).astype(jnp.float32)
        cnt_bt = jax.lax.dot_general(
            cnt, R, (((1,), (1,)), ((), ())),
            preferred_element_type=jnp.float32)
        live = cnt_bt > 0.5

        o_parts, m_parts, s_parts = [], [], []
        for h in range(H):
            khf = k_ref[:, :, h, :].reshape(T, D)
            vhf = v_ref[:, :, h, :].reshape(T, D)
            s_h = jax.lax.dot_general(
                q[:, h, :], khf, (((1,), (1,)), ((), ())),
                preferred_element_type=jnp.float32) * scale
            s_m = jnp.where(live, s_h, NEG_INF)
            m_h = jnp.max(s_m, axis=1, keepdims=True)
            p_h = jnp.exp(s_m - m_h) * cnt_bt
            s_h_sum = jnp.sum(p_h, axis=1, keepdims=True)
            o_h = jax.lax.dot_general(
                p_h, vhf, (((1,), (0,)), ((), ())),
                preferred_element_type=jnp.float32)
            o_parts.append(o_h[:, None, :])
            m_parts.append(m_h)
            s_parts.append(s_h_sum)

        o = jnp.concatenate(o_parts, axis=1)
        m = jnp.concatenate(m_parts, axis=1)
        s_sum = jnp.concatenate(s_parts, axis=1)

        send_ref[...] = jnp.concatenate(
            [o, m[:, :, None], s_sum[:, :, None]], axis=-1)

        rdma = pltpu.make_async_remote_copy(
            src_ref=send_ref, dst_ref=recv_ref,
            send_sem=send_sem, recv_sem=recv_sem,
            device_id=peer, device_id_type=pl.DeviceIdType.MESH)
        rdma.start()
        rdma.wait()

        r = recv_ref[...]
        o_p, m_p, s_p = r[:, :, :D], r[:, :, D], r[:, :, D + 1]
        m_tot = jnp.maximum(m, m_p)
        a = jnp.exp(m - m_tot)
        a_p = jnp.exp(m_p - m_tot)
        num = o * a[:, :, None] + o_p * a_p[:, :, None]
        den = s_sum * a + s_p * a_p
        out_ref[...] = num / den[:, :, None]

    out = pl.pallas_call(
        body,
        out_shape=jax.ShapeDtypeStruct((B, H, D), jnp.float32),
        in_specs=[pl.BlockSpec(memory_space=pltpu.VMEM)] * 5,
        out_specs=pl.BlockSpec(memory_space=pltpu.VMEM),
        scratch_shapes=[
            pltpu.VMEM((B, H, D + 2), jnp.float32),
            pltpu.VMEM((B, H, D + 2), jnp.float32),
            pltpu.SemaphoreType.DMA,
            pltpu.SemaphoreType.DMA,
        ],
        compiler_params=pltpu.CompilerParams(
            collective_id=0, vmem_limit_bytes=96 * 1024 * 1024),
    )(Q2, K, V, bt, lens2)
    return out.reshape(B, 1, H, D)
